# baseline (speedup 1.0000x reference)
import functools

import jax
import jax.numpy as jnp
from jax import lax
from jax.experimental import pallas as pl
from jax.experimental.pallas import tpu as pltpu
from jax.experimental.pallas import tpu_sc as plsc

N = 10000
D = 128
G = 64
NP = 10240
R = 640
NBLK = NP // R
NC, NS = 2, 16
RPT = NP // NS
CH = 88
NCHUNK = 116
PER_TILE = NCHUNK * CH
EP = NC * NS * PER_TILE
DEGW = 128

_f32 = jnp.float32


@functools.cache
def _mesh():
    return plsc.VectorSubcoreMesh(core_axis_name="c", subcore_axis_name="s",
                                  num_cores=NC, num_subcores=NS)


def _deg_call(dst_pad):
    def body(dst_hbm, out_hbm, idx_v, ones_v, zb, acc):
        cid = lax.axis_index("c")
        sid = lax.axis_index("s")
        for r in range(CH):
            for c in range(DEGW // 16):
                ones_v[r, c * 16:(c + 1) * 16] = jnp.ones((16,), _f32)
        for r in range(16):
            for c in range(DEGW // 16):
                zb[r, c * 16:(c + 1) * 16] = jnp.zeros((16,), _f32)
        rows0 = sid * RPT
        for j in range(RPT // 16):
            pltpu.sync_copy(zb, acc.at[pl.ds(rows0 + j * 16, 16)])
        plsc.subcore_barrier()
        tbase = (cid * NS + sid) * PER_TILE

        def step(t, c):
            pltpu.sync_copy(dst_hbm.at[pl.ds(tbase + t * CH, CH)], idx_v)
            pltpu.sync_copy(ones_v, acc.at[idx_v], add=True)
            return c

        lax.fori_loop(0, NCHUNK, step, 0)
        plsc.subcore_barrier()
        pltpu.sync_copy(acc.at[pl.ds(rows0, RPT)],
                        out_hbm.at[pl.ds(cid * NP + rows0, RPT)])

    return pl.kernel(
        body,
        out_type=jax.ShapeDtypeStruct((NC * NP, DEGW), _f32),
        mesh=_mesh(),
        scratch_types=[
            pltpu.VMEM((CH,), jnp.int32),
            pltpu.VMEM((CH, DEGW), _f32),
            pltpu.VMEM((16, DEGW), _f32),
            pltpu.VMEM_SHARED((NP, DEGW), _f32),
        ],
    )(dst_pad)


def _agg_call(src_pad, dst_pad, hp):
    def body(src_hbm, dst_hbm, hp_hbm, out_hbm,
             sb0, sb1, sb2, sb3, db0, db1, db2, db3,
             rb0, rb1, rb2, rb3, zb, acc, sm0, sm1, sm2, sm3):
        sb = (sb0, sb1, sb2, sb3)
        db = (db0, db1, db2, db3)
        rb = (rb0, rb1, rb2, rb3)
        sems = (sm0, sm1, sm2, sm3)
        cid = lax.axis_index("c")
        sid = lax.axis_index("s")
        for r in range(16):
            for c in range(8):
                zb[r, c * 16:(c + 1) * 16] = jnp.zeros((16,), _f32)
        rows0 = sid * RPT
        for j in range(RPT // 16):
            pltpu.sync_copy(zb, acc.at[pl.ds(rows0 + j * 16, 16)])
        plsc.subcore_barrier()
        tbase = (cid * NS + sid) * PER_TILE

        def start(k, t):
            pltpu.sync_copy(src_hbm.at[pl.ds(tbase + t * CH, CH)], sb[k])
            pltpu.async_copy(hp_hbm.at[sb[k]], rb[k], sems[k])

        def drain(k, t):
            pltpu.sync_copy(dst_hbm.at[pl.ds(tbase + t * CH, CH)], db[k])
            pltpu.make_async_copy(hp_hbm.at[sb[k]], rb[k], sems[k]).wait()
            pltpu.sync_copy(rb[k], acc.at[db[k]], add=True)

        start(0, 0)
        start(1, 1)

        def quad(q, c):
            t0 = 4 * q
            start(2, t0 + 2)
            start(3, t0 + 3)
            drain(0, t0)
            drain(1, t0 + 1)

            @pl.when(q < NCHUNK // 4 - 1)
            def _():
                start(0, t0 + 4)
                start(1, t0 + 5)

            drain(2, t0 + 2)
            drain(3, t0 + 3)
            return c

        lax.fori_loop(0, NCHUNK // 4, quad, 0)
        plsc.subcore_barrier()
        pltpu.sync_copy(acc.at[pl.ds(rows0, RPT)],
                        out_hbm.at[pl.ds(cid * NP + rows0, RPT)])

    return pl.kernel(
        body,
        out_type=jax.ShapeDtypeStruct((NC * NP, D), _f32),
        mesh=_mesh(),
        scratch_types=(
            [pltpu.VMEM((CH,), jnp.int32)] * 8
            + [pltpu.VMEM((CH, D), _f32)] * 4
            + [pltpu.VMEM((16, D), _f32),
               pltpu.VMEM_SHARED((NP, D), _f32)]
            + [pltpu.SemaphoreType.DMA] * 4
        ),
    )(src_pad, dst_pad, hp)


def _prep_body(x_ref, w_ref, d0_ref, d1_ref, o_ref):
    dinv = lax.rsqrt(d0_ref[:, 0] + d1_ref[:, 0] + 1.0)
    o_ref[...] = jnp.dot(x_ref[...], w_ref[...],
                         preferred_element_type=_f32) * dinv[:, None]


def _mid_body(p0_ref, p1_ref, hp_ref, d0_ref, d1_ref, b1_ref, w2_ref, o_ref):
    dinv = lax.rsqrt(d0_ref[:, 0] + d1_ref[:, 0] + 1.0)
    y = (p0_ref[...] + p1_ref[...] + hp_ref[...]) * dinv[:, None] + b1_ref[...]
    y = jnp.maximum(y, 0.0)
    o_ref[...] = jnp.dot(y, w2_ref[...],
                         preferred_element_type=_f32) * dinv[:, None]


def _final_body(q0_ref, q1_ref, hp_ref, d0_ref, d1_ref, b2_ref, bt_ref,
                wfc_ref, bfc_ref, o_ref, psum, csum):
    i = pl.program_id(0)

    @pl.when(i == 0)
    def _():
        psum[...] = jnp.zeros_like(psum)
        csum[...] = jnp.zeros_like(csum)

    dinv = lax.rsqrt(d0_ref[:, 0] + d1_ref[:, 0] + 1.0)
    h = (q0_ref[...] + q1_ref[...] + hp_ref[...]) * dinv[:, None] + b2_ref[...]
    rows = i * R + lax.broadcasted_iota(jnp.int32, (R, 1), 0)
    valid = rows < N
    h = jnp.where(valid, h, 0.0)
    bt = bt_ref[0, 0, :][:, None]
    gid = lax.broadcasted_iota(jnp.int32, (1, G), 1)
    onehot = (bt == gid).astype(_f32) * valid.astype(_f32)
    psum[...] += lax.dot_general(onehot, h, (((0,), (0,)), ((), ())),
                                 preferred_element_type=_f32)
    csum[...] += jnp.sum(onehot, axis=0)[:, None]
    pooled = psum[...] / jnp.maximum(csum[...], 1.0)
    o_ref[...] = jnp.dot(pooled, wfc_ref[...],
                         preferred_element_type=_f32) + bfc_ref[...]


def _row_spec(off):
    return pl.BlockSpec((R, D), lambda i, off=off: (i + off, 0))


def _deg_spec(off):
    return pl.BlockSpec((R, DEGW), lambda i, off=off: (i + off, 0))


def _full_spec(shape):
    nd = len(shape)
    return pl.BlockSpec(shape, lambda i: (0,) * nd)


def _prep_call(x, W1, deg2):
    return pl.pallas_call(
        _prep_body,
        grid=(NBLK,),
        in_specs=[_row_spec(0), _full_spec((D, D)), _deg_spec(0), _deg_spec(NBLK)],
        out_specs=_row_spec(0),
        out_shape=jax.ShapeDtypeStruct((N, D), _f32),
    )(x, W1, deg2, deg2)


def _mid_call(P, hp1, deg2, b1, W2):
    return pl.pallas_call(
        _mid_body,
        grid=(NBLK,),
        in_specs=[_row_spec(0), _row_spec(NBLK), _row_spec(0),
                  _deg_spec(0), _deg_spec(NBLK),
                  _full_spec((1, D)), _full_spec((D, D))],
        out_specs=_row_spec(0),
        out_shape=jax.ShapeDtypeStruct((N, D), _f32),
    )(P, P, hp1, deg2, deg2, b1, W2)


def _final_call(Q, hp2, deg2, b2, batch3, Wfc, bfc):
    return pl.pallas_call(
        _final_body,
        grid=(NBLK,),
        in_specs=[_row_spec(0), _row_spec(NBLK), _row_spec(0),
                  _deg_spec(0), _deg_spec(NBLK),
                  _full_spec((1, D)),
                  pl.BlockSpec((1, 1, R), lambda i: (i, 0, 0)),
                  _full_spec((D, D)), _full_spec((1, D))],
        out_specs=_full_spec((G, D)),
        out_shape=jax.ShapeDtypeStruct((G, D), _f32),
        scratch_shapes=[pltpu.VMEM((G, D), _f32), pltpu.VMEM((G, D), _f32)],
    )(Q, Q, hp2, deg2, deg2, b2, batch3, Wfc, bfc)


def kernel(x, edge_index, edge_attr, batch, W1, b1, W2, b2, Wfc, bfc):
    E = edge_index.shape[1]
    pad = EP - E
    ar = jnp.arange(pad, dtype=jnp.int32)
    src_pad = jnp.concatenate([edge_index[0], ar % 8192])
    dst_pad = jnp.concatenate([edge_index[1], N + (ar % CH)])
    batch3 = jnp.concatenate(
        [batch, jnp.full((NP - N,), G, jnp.int32)]).reshape(NBLK, 1, R)

    deg2 = _deg_call(dst_pad)
    hp1 = _prep_call(x, W1, deg2)
    P = _agg_call(src_pad, dst_pad, hp1)
    hp2 = _mid_call(P, hp1, deg2, b1.reshape(1, D), W2)
    Q = _agg_call(src_pad, dst_pad, hp2)
    return _final_call(Q, hp2, deg2, b2.reshape(1, D), batch3, Wfc,
                       bfc.reshape(1, D))


# speedup vs baseline: 1.1455x; 1.1455x over previous
import functools

import jax
import jax.numpy as jnp
from jax import lax
from jax.experimental import pallas as pl
from jax.experimental.pallas import tpu as pltpu
from jax.experimental.pallas import tpu_sc as plsc

N = 10000
D = 128
G = 64
NP = 10240
R = 640
NBLK = NP // R
NC, NS = 2, 16
RPT = NP // NS
CH = 128
NCHUNK = 80
PER_TILE = NCHUNK * CH
EP = NC * NS * PER_TILE
DEGW = 128

_f32 = jnp.float32


@functools.cache
def _mesh():
    return plsc.VectorSubcoreMesh(core_axis_name="c", subcore_axis_name="s",
                                  num_cores=NC, num_subcores=NS)


def _deg_call(dst_pad):
    def body(dst_hbm, out_hbm, idx_v, ones_v, zb, acc):
        cid = lax.axis_index("c")
        sid = lax.axis_index("s")
        for r in range(CH):
            for c in range(DEGW // 16):
                ones_v[r, c * 16:(c + 1) * 16] = jnp.ones((16,), _f32)
        for r in range(16):
            for c in range(DEGW // 16):
                zb[r, c * 16:(c + 1) * 16] = jnp.zeros((16,), _f32)
        rows0 = sid * RPT
        for j in range(RPT // 16):
            pltpu.sync_copy(zb, acc.at[pl.ds(rows0 + j * 16, 16)])
        plsc.subcore_barrier()
        tbase = (cid * NS + sid) * PER_TILE

        def step(t, c):
            pltpu.sync_copy(dst_hbm.at[pl.ds(tbase + t * CH, CH)], idx_v)
            pltpu.sync_copy(ones_v, acc.at[idx_v], add=True)
            return c

        lax.fori_loop(0, NCHUNK, step, 0)
        plsc.subcore_barrier()
        pltpu.sync_copy(acc.at[pl.ds(rows0, RPT)],
                        out_hbm.at[pl.ds(cid * NP + rows0, RPT)])

    return pl.kernel(
        body,
        out_type=jax.ShapeDtypeStruct((NC * NP, DEGW), _f32),
        mesh=_mesh(),
        scratch_types=[
            pltpu.VMEM((CH,), jnp.int32),
            pltpu.VMEM((CH, DEGW), _f32),
            pltpu.VMEM((16, DEGW), _f32),
            pltpu.VMEM_SHARED((NP, DEGW), _f32),
        ],
    )(dst_pad)


def _agg_call(src_pad, dst_pad, hp):
    def body(src_hbm, dst_hbm, hp_hbm, out_hbm,
             sb0, sb1, db0, db1, rb0, rb1, zb, acc, sg0, sg1, ss0, ss1):
        sb = (sb0, sb1)
        db = (db0, db1)
        rb = (rb0, rb1)
        sg = (sg0, sg1)
        ss = (ss0, ss1)
        cid = lax.axis_index("c")
        sid = lax.axis_index("s")
        for r in range(16):
            for c in range(8):
                zb[r, c * 16:(c + 1) * 16] = jnp.zeros((16,), _f32)
        rows0 = sid * RPT
        for j in range(RPT // 16):
            pltpu.sync_copy(zb, acc.at[pl.ds(rows0 + j * 16, 16)])
        plsc.subcore_barrier()
        tbase = (cid * NS + sid) * PER_TILE

        def gstart(k, t):
            pltpu.sync_copy(src_hbm.at[pl.ds(tbase + t * CH, CH)], sb[k])
            pltpu.async_copy(hp_hbm.at[sb[k]], rb[k], sg[k])

        def gwait(k):
            pltpu.make_async_copy(hp_hbm.at[sb[k]], rb[k], sg[k]).wait()

        def sstart(k, t):
            pltpu.sync_copy(dst_hbm.at[pl.ds(tbase + t * CH, CH)], db[k])
            pltpu.make_async_copy(rb[k], acc.at[db[k]], ss[k]).start(add=True)

        def swait(k):
            pltpu.make_async_copy(rb[k], acc.at[db[k]], ss[k]).wait()

        gstart(0, 0)
        gstart(1, 1)

        def pair(p, c):
            gwait(0)
            sstart(0, 2 * p)
            gwait(1)
            sstart(1, 2 * p + 1)

            @pl.when(p < NCHUNK // 2 - 1)
            def _():
                swait(0)
                gstart(0, 2 * p + 2)
                swait(1)
                gstart(1, 2 * p + 3)

            return c

        lax.fori_loop(0, NCHUNK // 2, pair, 0)
        swait(0)
        swait(1)
        plsc.subcore_barrier()
        pltpu.sync_copy(acc.at[pl.ds(rows0, RPT)],
                        out_hbm.at[pl.ds(cid * NP + rows0, RPT)])

    return pl.kernel(
        body,
        out_type=jax.ShapeDtypeStruct((NC * NP, D), _f32),
        mesh=_mesh(),
        scratch_types=(
            [pltpu.VMEM((CH,), jnp.int32)] * 4
            + [pltpu.VMEM((CH, D), _f32)] * 2
            + [pltpu.VMEM((16, D), _f32),
               pltpu.VMEM_SHARED((NP, D), _f32)]
            + [pltpu.SemaphoreType.DMA] * 4
        ),
    )(src_pad, dst_pad, hp)


def _prep_body(x_ref, w_ref, d0_ref, d1_ref, o_ref):
    dinv = lax.rsqrt(d0_ref[:, 0] + d1_ref[:, 0] + 1.0)
    o_ref[...] = jnp.dot(x_ref[...], w_ref[...],
                         preferred_element_type=_f32) * dinv[:, None]


def _mid_body(p0_ref, p1_ref, hp_ref, d0_ref, d1_ref, b1_ref, w2_ref, o_ref):
    dinv = lax.rsqrt(d0_ref[:, 0] + d1_ref[:, 0] + 1.0)
    y = (p0_ref[...] + p1_ref[...] + hp_ref[...]) * dinv[:, None] + b1_ref[...]
    y = jnp.maximum(y, 0.0)
    o_ref[...] = jnp.dot(y, w2_ref[...],
                         preferred_element_type=_f32) * dinv[:, None]


def _final_body(q0_ref, q1_ref, hp_ref, d0_ref, d1_ref, b2_ref, bt_ref,
                wfc_ref, bfc_ref, o_ref, psum, csum):
    i = pl.program_id(0)

    @pl.when(i == 0)
    def _():
        psum[...] = jnp.zeros_like(psum)
        csum[...] = jnp.zeros_like(csum)

    dinv = lax.rsqrt(d0_ref[:, 0] + d1_ref[:, 0] + 1.0)
    h = (q0_ref[...] + q1_ref[...] + hp_ref[...]) * dinv[:, None] + b2_ref[...]
    rows = i * R + lax.broadcasted_iota(jnp.int32, (R, 1), 0)
    valid = rows < N
    h = jnp.where(valid, h, 0.0)
    bt = bt_ref[0, 0, :][:, None]
    gid = lax.broadcasted_iota(jnp.int32, (1, G), 1)
    onehot = (bt == gid).astype(_f32) * valid.astype(_f32)
    psum[...] += lax.dot_general(onehot, h, (((0,), (0,)), ((), ())),
                                 preferred_element_type=_f32)
    csum[...] += jnp.sum(onehot, axis=0)[:, None]
    pooled = psum[...] / jnp.maximum(csum[...], 1.0)
    o_ref[...] = jnp.dot(pooled, wfc_ref[...],
                         preferred_element_type=_f32) + bfc_ref[...]


def _row_spec(off):
    return pl.BlockSpec((R, D), lambda i, off=off: (i + off, 0))


def _deg_spec(off):
    return pl.BlockSpec((R, DEGW), lambda i, off=off: (i + off, 0))


def _full_spec(shape):
    nd = len(shape)
    return pl.BlockSpec(shape, lambda i: (0,) * nd)


def _prep_call(x, W1, deg2):
    return pl.pallas_call(
        _prep_body,
        grid=(NBLK,),
        in_specs=[_row_spec(0), _full_spec((D, D)), _deg_spec(0), _deg_spec(NBLK)],
        out_specs=_row_spec(0),
        out_shape=jax.ShapeDtypeStruct((N, D), _f32),
    )(x, W1, deg2, deg2)


def _mid_call(P, hp1, deg2, b1, W2):
    return pl.pallas_call(
        _mid_body,
        grid=(NBLK,),
        in_specs=[_row_spec(0), _row_spec(NBLK), _row_spec(0),
                  _deg_spec(0), _deg_spec(NBLK),
                  _full_spec((1, D)), _full_spec((D, D))],
        out_specs=_row_spec(0),
        out_shape=jax.ShapeDtypeStruct((N, D), _f32),
    )(P, P, hp1, deg2, deg2, b1, W2)


def _final_call(Q, hp2, deg2, b2, batch3, Wfc, bfc):
    return pl.pallas_call(
        _final_body,
        grid=(NBLK,),
        in_specs=[_row_spec(0), _row_spec(NBLK), _row_spec(0),
                  _deg_spec(0), _deg_spec(NBLK),
                  _full_spec((1, D)),
                  pl.BlockSpec((1, 1, R), lambda i: (i, 0, 0)),
                  _full_spec((D, D)), _full_spec((1, D))],
        out_specs=_full_spec((G, D)),
        out_shape=jax.ShapeDtypeStruct((G, D), _f32),
        scratch_shapes=[pltpu.VMEM((G, D), _f32), pltpu.VMEM((G, D), _f32)],
    )(Q, Q, hp2, deg2, deg2, b2, batch3, Wfc, bfc)


def kernel(x, edge_index, edge_attr, batch, W1, b1, W2, b2, Wfc, bfc):
    E = edge_index.shape[1]
    pad = EP - E
    ar = jnp.arange(pad, dtype=jnp.int32)
    src_pad = jnp.concatenate([edge_index[0], ar % 8192])
    dst_pad = jnp.concatenate([edge_index[1], N + (ar % CH)])
    batch3 = jnp.concatenate(
        [batch, jnp.full((NP - N,), G, jnp.int32)]).reshape(NBLK, 1, R)

    deg2 = _deg_call(dst_pad)
    hp1 = _prep_call(x, W1, deg2)
    P = _agg_call(src_pad, dst_pad, hp1)
    hp2 = _mid_call(P, hp1, deg2, b1.reshape(1, D), W2)
    Q = _agg_call(src_pad, dst_pad, hp2)
    return _final_call(Q, hp2, deg2, b2.reshape(1, D), batch3, Wfc,
                       bfc.reshape(1, D))


# speedup vs baseline: 1.2335x; 1.0768x over previous
import functools

import jax
import jax.numpy as jnp
from jax import lax
from jax.experimental import pallas as pl
from jax.experimental.pallas import tpu as pltpu
from jax.experimental.pallas import tpu_sc as plsc

N = 10000
D = 128
G = 64
NP = 10240
R = 640
NBLK = NP // R
NC, NS = 2, 16
RPT = NP // NS
CH = 128
NCHUNK = 80
PER_TILE = NCHUNK * CH
EP = NC * NS * PER_TILE
DEGW = 128

_f32 = jnp.float32


@functools.cache
def _mesh():
    return plsc.VectorSubcoreMesh(core_axis_name="c", subcore_axis_name="s",
                                  num_cores=NC, num_subcores=NS)


def _deg_call(dst_pad, w=DEGW):
    def body(dst_hbm, out_hbm, db0, db1, ones_v, zb, acc, ss0, ss1):
        db = (db0, db1)
        ss = (ss0, ss1)
        cid = lax.axis_index("c")
        sid = lax.axis_index("s")
        for r in range(CH):
            for c in range(w // 16):
                ones_v[r, c * 16:(c + 1) * 16] = jnp.ones((16,), _f32)
        for r in range(16):
            for c in range(w // 16):
                zb[r, c * 16:(c + 1) * 16] = jnp.zeros((16,), _f32)
        rows0 = sid * RPT
        for j in range(RPT // 16):
            pltpu.sync_copy(zb, acc.at[pl.ds(rows0 + j * 16, 16)])
        plsc.subcore_barrier()
        tbase = (cid * NS + sid) * PER_TILE

        def sstart(k, t):
            pltpu.sync_copy(dst_hbm.at[pl.ds(tbase + t * CH, CH)], db[k])
            pltpu.make_async_copy(ones_v, acc.at[db[k]], ss[k]).start(add=True)

        def swait(k):
            pltpu.make_async_copy(ones_v, acc.at[db[k]], ss[k]).wait()

        sstart(0, 0)
        sstart(1, 1)

        def pair(p, c):
            @pl.when(p < NCHUNK // 2 - 1)
            def _():
                swait(0)
                sstart(0, 2 * p + 2)
                swait(1)
                sstart(1, 2 * p + 3)

            return c

        lax.fori_loop(0, NCHUNK // 2, pair, 0)
        swait(0)
        swait(1)
        plsc.subcore_barrier()
        pltpu.sync_copy(acc.at[pl.ds(rows0, RPT)],
                        out_hbm.at[pl.ds(cid * NP + rows0, RPT)])

    return pl.kernel(
        body,
        out_type=jax.ShapeDtypeStruct((NC * NP, w), _f32),
        mesh=_mesh(),
        scratch_types=[
            pltpu.VMEM((CH,), jnp.int32),
            pltpu.VMEM((CH,), jnp.int32),
            pltpu.VMEM((CH, w), _f32),
            pltpu.VMEM((16, w), _f32),
            pltpu.VMEM_SHARED((NP, w), _f32),
            pltpu.SemaphoreType.DMA,
            pltpu.SemaphoreType.DMA,
        ],
    )(dst_pad)


def _agg_call(src_pad, dst_pad, hp):
    def body(src_hbm, dst_hbm, hp_hbm, out_hbm,
             sb0, sb1, db0, db1, rb0, rb1, zb, acc, sg0, sg1, ss0, ss1):
        sb = (sb0, sb1)
        db = (db0, db1)
        rb = (rb0, rb1)
        sg = (sg0, sg1)
        ss = (ss0, ss1)
        cid = lax.axis_index("c")
        sid = lax.axis_index("s")
        for r in range(16):
            for c in range(8):
                zb[r, c * 16:(c + 1) * 16] = jnp.zeros((16,), _f32)
        rows0 = sid * RPT
        for j in range(RPT // 16):
            pltpu.sync_copy(zb, acc.at[pl.ds(rows0 + j * 16, 16)])
        plsc.subcore_barrier()
        tbase = (cid * NS + sid) * PER_TILE

        def gstart(k, t):
            pltpu.sync_copy(src_hbm.at[pl.ds(tbase + t * CH, CH)], sb[k])
            pltpu.async_copy(hp_hbm.at[sb[k]], rb[k], sg[k])

        def gwait(k):
            pltpu.make_async_copy(hp_hbm.at[sb[k]], rb[k], sg[k]).wait()

        def sstart(k, t):
            pltpu.sync_copy(dst_hbm.at[pl.ds(tbase + t * CH, CH)], db[k])
            pltpu.make_async_copy(rb[k], acc.at[db[k]], ss[k]).start(add=True)

        def swait(k):
            pltpu.make_async_copy(rb[k], acc.at[db[k]], ss[k]).wait()

        gstart(0, 0)
        gstart(1, 1)

        def pair(p, c):
            gwait(0)
            sstart(0, 2 * p)
            gwait(1)
            sstart(1, 2 * p + 1)

            @pl.when(p < NCHUNK // 2 - 1)
            def _():
                swait(0)
                gstart(0, 2 * p + 2)
                swait(1)
                gstart(1, 2 * p + 3)

            return c

        lax.fori_loop(0, NCHUNK // 2, pair, 0)
        swait(0)
        swait(1)
        plsc.subcore_barrier()
        pltpu.sync_copy(acc.at[pl.ds(rows0, RPT)],
                        out_hbm.at[pl.ds(cid * NP + rows0, RPT)])

    return pl.kernel(
        body,
        out_type=jax.ShapeDtypeStruct((NC * NP, D), _f32),
        mesh=_mesh(),
        scratch_types=(
            [pltpu.VMEM((CH,), jnp.int32)] * 4
            + [pltpu.VMEM((CH, D), _f32)] * 2
            + [pltpu.VMEM((16, D), _f32),
               pltpu.VMEM_SHARED((NP, D), _f32)]
            + [pltpu.SemaphoreType.DMA] * 4
        ),
    )(src_pad, dst_pad, hp)


def _prep_body(x_ref, w_ref, d0_ref, d1_ref, o_ref):
    dinv = lax.rsqrt(d0_ref[:, 0] + d1_ref[:, 0] + 1.0)
    o_ref[...] = jnp.dot(x_ref[...], w_ref[...],
                         preferred_element_type=_f32) * dinv[:, None]


def _mid_body(p0_ref, p1_ref, hp_ref, d0_ref, d1_ref, b1_ref, w2_ref, o_ref):
    dinv = lax.rsqrt(d0_ref[:, 0] + d1_ref[:, 0] + 1.0)
    y = (p0_ref[...] + p1_ref[...] + hp_ref[...]) * dinv[:, None] + b1_ref[...]
    y = jnp.maximum(y, 0.0)
    o_ref[...] = jnp.dot(y, w2_ref[...],
                         preferred_element_type=_f32) * dinv[:, None]


def _final_body(q0_ref, q1_ref, hp_ref, d0_ref, d1_ref, b2_ref, bt_ref,
                wfc_ref, bfc_ref, o_ref, psum, csum):
    i = pl.program_id(0)

    @pl.when(i == 0)
    def _():
        psum[...] = jnp.zeros_like(psum)
        csum[...] = jnp.zeros_like(csum)

    dinv = lax.rsqrt(d0_ref[:, 0] + d1_ref[:, 0] + 1.0)
    h = (q0_ref[...] + q1_ref[...] + hp_ref[...]) * dinv[:, None] + b2_ref[...]
    rows = i * R + lax.broadcasted_iota(jnp.int32, (R, 1), 0)
    valid = rows < N
    h = jnp.where(valid, h, 0.0)
    bt = bt_ref[0, 0, :][:, None]
    gid = lax.broadcasted_iota(jnp.int32, (1, G), 1)
    onehot = (bt == gid).astype(_f32) * valid.astype(_f32)
    psum[...] += lax.dot_general(onehot, h, (((0,), (0,)), ((), ())),
                                 preferred_element_type=_f32)
    csum[...] += jnp.sum(onehot, axis=0)[:, None]
    pooled = psum[...] / jnp.maximum(csum[...], 1.0)
    o_ref[...] = jnp.dot(pooled, wfc_ref[...],
                         preferred_element_type=_f32) + bfc_ref[...]


def _row_spec(off):
    return pl.BlockSpec((R, D), lambda i, off=off: (i + off, 0))


def _deg_spec(off):
    return pl.BlockSpec((R, DEGW), lambda i, off=off: (i + off, 0))


def _full_spec(shape):
    nd = len(shape)
    return pl.BlockSpec(shape, lambda i: (0,) * nd)


def _prep_call(x, W1, deg2):
    return pl.pallas_call(
        _prep_body,
        grid=(NBLK,),
        in_specs=[_row_spec(0), _full_spec((D, D)), _deg_spec(0), _deg_spec(NBLK)],
        out_specs=_row_spec(0),
        out_shape=jax.ShapeDtypeStruct((N, D), _f32),
    )(x, W1, deg2, deg2)


def _mid_call(P, hp1, deg2, b1, W2):
    return pl.pallas_call(
        _mid_body,
        grid=(NBLK,),
        in_specs=[_row_spec(0), _row_spec(NBLK), _row_spec(0),
                  _deg_spec(0), _deg_spec(NBLK),
                  _full_spec((1, D)), _full_spec((D, D))],
        out_specs=_row_spec(0),
        out_shape=jax.ShapeDtypeStruct((N, D), _f32),
    )(P, P, hp1, deg2, deg2, b1, W2)


def _final_call(Q, hp2, deg2, b2, batch3, Wfc, bfc):
    return pl.pallas_call(
        _final_body,
        grid=(NBLK,),
        in_specs=[_row_spec(0), _row_spec(NBLK), _row_spec(0),
                  _deg_spec(0), _deg_spec(NBLK),
                  _full_spec((1, D)),
                  pl.BlockSpec((1, 1, R), lambda i: (i, 0, 0)),
                  _full_spec((D, D)), _full_spec((1, D))],
        out_specs=_full_spec((G, D)),
        out_shape=jax.ShapeDtypeStruct((G, D), _f32),
        scratch_shapes=[pltpu.VMEM((G, D), _f32), pltpu.VMEM((G, D), _f32)],
    )(Q, Q, hp2, deg2, deg2, b2, batch3, Wfc, bfc)


def kernel(x, edge_index, edge_attr, batch, W1, b1, W2, b2, Wfc, bfc):
    E = edge_index.shape[1]
    pad = EP - E
    ar = jnp.arange(pad, dtype=jnp.int32)
    src_pad = jnp.concatenate([edge_index[0], ar % 8192])
    dst_pad = jnp.concatenate([edge_index[1], N + (ar % CH)])
    batch3 = jnp.concatenate(
        [batch, jnp.full((NP - N,), G, jnp.int32)]).reshape(NBLK, 1, R)

    deg2 = _deg_call(dst_pad)
    hp1 = _prep_call(x, W1, deg2)
    P = _agg_call(src_pad, dst_pad, hp1)
    hp2 = _mid_call(P, hp1, deg2, b1.reshape(1, D), W2)
    Q = _agg_call(src_pad, dst_pad, hp2)
    return _final_call(Q, hp2, deg2, b2.reshape(1, D), batch3, Wfc,
                       bfc.reshape(1, D))


# speedup vs baseline: 1.2355x; 1.0016x over previous
import functools

import jax
import jax.numpy as jnp
from jax import lax
from jax.experimental import pallas as pl
from jax.experimental.pallas import tpu as pltpu
from jax.experimental.pallas import tpu_sc as plsc

N = 10000
D = 128
G = 64
NP = 10240
R = 640
NBLK = NP // R
NC, NS = 2, 16
RPT = NP // NS
CH = 128
NCHUNK = 80
PER_TILE = NCHUNK * CH
EP = NC * NS * PER_TILE
DEGW = 128
DEGOUT = 16

_f32 = jnp.float32


@functools.cache
def _mesh():
    return plsc.VectorSubcoreMesh(core_axis_name="c", subcore_axis_name="s",
                                  num_cores=NC, num_subcores=NS)


def _deg_call(dst_pad, w=DEGW):
    def body(dst_hbm, out_hbm, db0, db1, ones_v, zb, acc, ss0, ss1):
        db = (db0, db1)
        ss = (ss0, ss1)
        cid = lax.axis_index("c")
        sid = lax.axis_index("s")
        for r in range(CH):
            for c in range(w // 16):
                ones_v[r, c * 16:(c + 1) * 16] = jnp.ones((16,), _f32)
        for r in range(16):
            for c in range(w // 16):
                zb[r, c * 16:(c + 1) * 16] = jnp.zeros((16,), _f32)
        rows0 = sid * RPT
        for j in range(RPT // 16):
            pltpu.sync_copy(zb, acc.at[pl.ds(rows0 + j * 16, 16)])
        plsc.subcore_barrier()
        tbase = (cid * NS + sid) * PER_TILE

        def sstart(k, t):
            pltpu.sync_copy(dst_hbm.at[pl.ds(tbase + t * CH, CH)], db[k])
            pltpu.make_async_copy(ones_v, acc.at[db[k]], ss[k]).start(add=True)

        def swait(k):
            pltpu.make_async_copy(ones_v, acc.at[db[k]], ss[k]).wait()

        sstart(0, 0)
        sstart(1, 1)

        def pair(p, c):
            @pl.when(p < NCHUNK // 2 - 1)
            def _():
                swait(0)
                sstart(0, 2 * p + 2)
                swait(1)
                sstart(1, 2 * p + 3)

            return c

        lax.fori_loop(0, NCHUNK // 2, pair, 0)
        swait(0)
        swait(1)
        plsc.subcore_barrier()
        pltpu.sync_copy(acc.at[pl.ds(rows0, RPT)],
                        out_hbm.at[pl.ds(cid * NP + rows0, RPT)])

    return pl.kernel(
        body,
        out_type=jax.ShapeDtypeStruct((NC * NP, w), _f32),
        mesh=_mesh(),
        scratch_types=[
            pltpu.VMEM((CH,), jnp.int32),
            pltpu.VMEM((CH,), jnp.int32),
            pltpu.VMEM((CH, w), _f32),
            pltpu.VMEM((16, w), _f32),
            pltpu.VMEM_SHARED((NP, w), _f32),
            pltpu.SemaphoreType.DMA,
            pltpu.SemaphoreType.DMA,
        ],
    )(dst_pad)


def _agg_call(src_pad, dst_pad, hp):
    def body(src_hbm, dst_hbm, hp_hbm, out_hbm,
             sb0, sb1, db0, db1, rb0, rb1, zb, acc, sg0, sg1, ss0, ss1):
        sb = (sb0, sb1)
        db = (db0, db1)
        rb = (rb0, rb1)
        sg = (sg0, sg1)
        ss = (ss0, ss1)
        cid = lax.axis_index("c")
        sid = lax.axis_index("s")
        for r in range(16):
            for c in range(8):
                zb[r, c * 16:(c + 1) * 16] = jnp.zeros((16,), _f32)
        rows0 = sid * RPT
        for j in range(RPT // 16):
            pltpu.sync_copy(zb, acc.at[pl.ds(rows0 + j * 16, 16)])
        plsc.subcore_barrier()
        tbase = (cid * NS + sid) * PER_TILE

        def gstart(k, t):
            pltpu.sync_copy(src_hbm.at[pl.ds(tbase + t * CH, CH)], sb[k])
            pltpu.async_copy(hp_hbm.at[sb[k]], rb[k], sg[k])

        def gwait(k):
            pltpu.make_async_copy(hp_hbm.at[sb[k]], rb[k], sg[k]).wait()

        def sstart(k, t):
            pltpu.sync_copy(dst_hbm.at[pl.ds(tbase + t * CH, CH)], db[k])
            pltpu.make_async_copy(rb[k], acc.at[db[k]], ss[k]).start(add=True)

        def swait(k):
            pltpu.make_async_copy(rb[k], acc.at[db[k]], ss[k]).wait()

        gstart(0, 0)
        gstart(1, 1)

        def pair(p, c):
            gwait(0)
            sstart(0, 2 * p)
            gwait(1)
            sstart(1, 2 * p + 1)

            @pl.when(p < NCHUNK // 2 - 1)
            def _():
                swait(0)
                gstart(0, 2 * p + 2)
                swait(1)
                gstart(1, 2 * p + 3)

            return c

        lax.fori_loop(0, NCHUNK // 2, pair, 0)
        swait(0)
        swait(1)
        plsc.subcore_barrier()
        pltpu.sync_copy(acc.at[pl.ds(rows0, RPT)],
                        out_hbm.at[pl.ds(cid * NP + rows0, RPT)])

    return pl.kernel(
        body,
        out_type=jax.ShapeDtypeStruct((NC * NP, D), _f32),
        mesh=_mesh(),
        scratch_types=(
            [pltpu.VMEM((CH,), jnp.int32)] * 4
            + [pltpu.VMEM((CH, D), _f32)] * 2
            + [pltpu.VMEM((16, D), _f32),
               pltpu.VMEM_SHARED((NP, D), _f32)]
            + [pltpu.SemaphoreType.DMA] * 4
        ),
    )(src_pad, dst_pad, hp)


def _prep_body(x_ref, w_ref, d0_ref, d1_ref, o_ref, dv_ref):
    dinv = lax.rsqrt(d0_ref[:, 0] + d1_ref[:, 0] + 1.0)
    dv_ref[...] = dinv[:, None] * jnp.ones((1, DEGOUT), _f32)
    o_ref[...] = jnp.dot(x_ref[...], w_ref[...],
                         preferred_element_type=_f32) * dinv[:, None]


def _mid_body(p0_ref, p1_ref, hp_ref, dv_ref, b1_ref, w2_ref, o_ref):
    dinv = dv_ref[:, 0]
    y = (p0_ref[...] + p1_ref[...] + hp_ref[...]) * dinv[:, None] + b1_ref[...]
    y = jnp.maximum(y, 0.0)
    o_ref[...] = jnp.dot(y, w2_ref[...],
                         preferred_element_type=_f32) * dinv[:, None]


def _final_body(q0_ref, q1_ref, hp_ref, dv_ref, b2_ref, bt_ref,
                wfc_ref, bfc_ref, o_ref, psum, csum):
    i = pl.program_id(0)

    @pl.when(i == 0)
    def _():
        psum[...] = jnp.zeros_like(psum)
        csum[...] = jnp.zeros_like(csum)

    dinv = dv_ref[:, 0]
    h = (q0_ref[...] + q1_ref[...] + hp_ref[...]) * dinv[:, None] + b2_ref[...]
    rows = i * R + lax.broadcasted_iota(jnp.int32, (R, 1), 0)
    valid = rows < N
    h = jnp.where(valid, h, 0.0)
    bt = bt_ref[0, 0, :][:, None]
    gid = lax.broadcasted_iota(jnp.int32, (1, G), 1)
    onehot = (bt == gid).astype(_f32) * valid.astype(_f32)
    psum[...] += lax.dot_general(onehot, h, (((0,), (0,)), ((), ())),
                                 preferred_element_type=_f32)
    csum[...] += jnp.sum(onehot, axis=0)[:, None]
    pooled = psum[...] / jnp.maximum(csum[...], 1.0)
    o_ref[...] = jnp.dot(pooled, wfc_ref[...],
                         preferred_element_type=_f32) + bfc_ref[...]


def _row_spec(off):
    return pl.BlockSpec((R, D), lambda i, off=off: (i + off, 0))


def _deg_spec(off):
    return pl.BlockSpec((R, DEGW), lambda i, off=off: (i + off, 0))


def _full_spec(shape):
    nd = len(shape)
    return pl.BlockSpec(shape, lambda i: (0,) * nd)


def _dinv_spec():
    return pl.BlockSpec((R, DEGOUT), lambda i: (i, 0))


def _prep_call(x, W1, deg2):
    return pl.pallas_call(
        _prep_body,
        grid=(NBLK,),
        in_specs=[_row_spec(0), _full_spec((D, D)), _deg_spec(0), _deg_spec(NBLK)],
        out_specs=(_row_spec(0), _dinv_spec()),
        out_shape=(jax.ShapeDtypeStruct((N, D), _f32),
                   jax.ShapeDtypeStruct((NP, DEGOUT), _f32)),
    )(x, W1, deg2, deg2)


def _mid_call(P, hp1, dinv, b1, W2):
    return pl.pallas_call(
        _mid_body,
        grid=(NBLK,),
        in_specs=[_row_spec(0), _row_spec(NBLK), _row_spec(0),
                  _dinv_spec(),
                  _full_spec((1, D)), _full_spec((D, D))],
        out_specs=_row_spec(0),
        out_shape=jax.ShapeDtypeStruct((N, D), _f32),
    )(P, P, hp1, dinv, b1, W2)


def _final_call(Q, hp2, dinv, b2, batch3, Wfc, bfc):
    return pl.pallas_call(
        _final_body,
        grid=(NBLK,),
        in_specs=[_row_spec(0), _row_spec(NBLK), _row_spec(0),
                  _dinv_spec(),
                  _full_spec((1, D)),
                  pl.BlockSpec((1, 1, R), lambda i: (i, 0, 0)),
                  _full_spec((D, D)), _full_spec((1, D))],
        out_specs=_full_spec((G, D)),
        out_shape=jax.ShapeDtypeStruct((G, D), _f32),
        scratch_shapes=[pltpu.VMEM((G, D), _f32), pltpu.VMEM((G, D), _f32)],
    )(Q, Q, hp2, dinv, b2, batch3, Wfc, bfc)


def kernel(x, edge_index, edge_attr, batch, W1, b1, W2, b2, Wfc, bfc):
    E = edge_index.shape[1]
    pad = EP - E
    ar = jnp.arange(pad, dtype=jnp.int32)
    src_pad = jnp.concatenate([edge_index[0], ar % 8192])
    dst_pad = jnp.concatenate([edge_index[1], N + (ar % CH)])
    batch3 = jnp.concatenate(
        [batch, jnp.full((NP - N,), G, jnp.int32)]).reshape(NBLK, 1, R)

    deg2 = _deg_call(dst_pad)
    hp1, dinv = _prep_call(x, W1, deg2)
    P = _agg_call(src_pad, dst_pad, hp1)
    hp2 = _mid_call(P, hp1, dinv, b1.reshape(1, D), W2)
    Q = _agg_call(src_pad, dst_pad, hp2)
    return _final_call(Q, hp2, dinv, b2.reshape(1, D), batch3, Wfc,
                       bfc.reshape(1, D))


# speedup vs baseline: 1.4304x; 1.1577x over previous
import functools

import jax
import jax.numpy as jnp
from jax import lax
from jax.experimental import pallas as pl
from jax.experimental.pallas import tpu as pltpu
from jax.experimental.pallas import tpu_sc as plsc

N = 10000
D = 128
G = 64
NP = 10240
R = 640
NBLK = NP // R
NC, NS = 2, 16
RPT = NP // NS
CH = 128
NCHUNK = 80
PER_TILE = NCHUNK * CH
EP = NC * NS * PER_TILE
DEGW = 128
DEGOUT = 16

_f32 = jnp.float32


@functools.cache
def _mesh():
    return plsc.VectorSubcoreMesh(core_axis_name="c", subcore_axis_name="s",
                                  num_cores=NC, num_subcores=NS)


def _deg_call(dst_pad):
    def body(dst_hbm, out_hbm, idx, hist, red, outv, staging):
        cid = lax.axis_index("c")
        sid = lax.axis_index("s")
        ones = jnp.ones((16,), _f32)
        zeros = jnp.zeros((16,), _f32)

        def zstep(t, c):
            hist[pl.ds(t * 16, 16)] = zeros
            return c

        lax.fori_loop(0, NP // 16, zstep, 0)
        tbase = (cid * NS + sid) * PER_TILE
        pltpu.sync_copy(dst_hbm.at[pl.ds(tbase, PER_TILE)], idx)

        def hstep(t, c):
            iv = idx[pl.ds(t * 16, 16)]
            plsc.addupdate_scatter(hist, [iv], ones)
            return c

        lax.fori_loop(0, PER_TILE // 16, hstep, 0)
        pltpu.sync_copy(hist, staging.at[sid])
        plsc.subcore_barrier()
        pltpu.sync_copy(staging.at[:, pl.ds(sid * RPT, RPT)], red)
        for j in range(RPT // 16):
            s = red[0, j * 16:(j + 1) * 16]
            for r in range(1, NS):
                s = s + red[r, j * 16:(j + 1) * 16]
            outv[j * 16:(j + 1) * 16] = s
        pltpu.sync_copy(outv, out_hbm.at[pl.ds(cid * NP + sid * RPT, RPT)])

    return pl.kernel(
        body,
        out_type=jax.ShapeDtypeStruct((NC * NP,), _f32),
        mesh=_mesh(),
        compiler_params=pltpu.CompilerParams(needs_layout_passes=False),
        scratch_types=[
            pltpu.VMEM((PER_TILE,), jnp.int32),
            pltpu.VMEM((NP,), _f32),
            pltpu.VMEM((NS, RPT), _f32),
            pltpu.VMEM((RPT,), _f32),
            pltpu.VMEM_SHARED((NS, NP), _f32),
        ],
    )(dst_pad)


def _agg_call(src_pad, dst_pad, hp):
    def body(src_hbm, dst_hbm, hp_hbm, out_hbm,
             sb0, sb1, db0, db1, rb0, rb1, zb, acc, sg0, sg1, ss0, ss1):
        sb = (sb0, sb1)
        db = (db0, db1)
        rb = (rb0, rb1)
        sg = (sg0, sg1)
        ss = (ss0, ss1)
        cid = lax.axis_index("c")
        sid = lax.axis_index("s")
        for r in range(16):
            for c in range(8):
                zb[r, c * 16:(c + 1) * 16] = jnp.zeros((16,), _f32)
        rows0 = sid * RPT
        for j in range(RPT // 16):
            pltpu.sync_copy(zb, acc.at[pl.ds(rows0 + j * 16, 16)])
        plsc.subcore_barrier()
        tbase = (cid * NS + sid) * PER_TILE

        def gstart(k, t):
            pltpu.sync_copy(src_hbm.at[pl.ds(tbase + t * CH, CH)], sb[k])
            pltpu.async_copy(hp_hbm.at[sb[k]], rb[k], sg[k])

        def gwait(k):
            pltpu.make_async_copy(hp_hbm.at[sb[k]], rb[k], sg[k]).wait()

        def sstart(k, t):
            pltpu.sync_copy(dst_hbm.at[pl.ds(tbase + t * CH, CH)], db[k])
            pltpu.make_async_copy(rb[k], acc.at[db[k]], ss[k]).start(add=True)

        def swait(k):
            pltpu.make_async_copy(rb[k], acc.at[db[k]], ss[k]).wait()

        gstart(0, 0)
        gstart(1, 1)

        def pair(p, c):
            gwait(0)
            sstart(0, 2 * p)
            gwait(1)
            sstart(1, 2 * p + 1)

            @pl.when(p < NCHUNK // 2 - 1)
            def _():
                swait(0)
                gstart(0, 2 * p + 2)
                swait(1)
                gstart(1, 2 * p + 3)

            return c

        lax.fori_loop(0, NCHUNK // 2, pair, 0)
        swait(0)
        swait(1)
        plsc.subcore_barrier()
        pltpu.sync_copy(acc.at[pl.ds(rows0, RPT)],
                        out_hbm.at[pl.ds(cid * NP + rows0, RPT)])

    return pl.kernel(
        body,
        out_type=jax.ShapeDtypeStruct((NC * NP, D), _f32),
        mesh=_mesh(),
        scratch_types=(
            [pltpu.VMEM((CH,), jnp.int32)] * 4
            + [pltpu.VMEM((CH, D), _f32)] * 2
            + [pltpu.VMEM((16, D), _f32),
               pltpu.VMEM_SHARED((NP, D), _f32)]
            + [pltpu.SemaphoreType.DMA] * 4
        ),
    )(src_pad, dst_pad, hp)


def _prep_body(x_ref, w_ref, d0_ref, d1_ref, o_ref, dv_ref):
    dinv = lax.rsqrt(d0_ref[0, 0, :] + d1_ref[0, 0, :] + 1.0)
    dv_ref[...] = dinv[:, None] * jnp.ones((1, DEGOUT), _f32)
    o_ref[...] = jnp.dot(x_ref[...], w_ref[...],
                         preferred_element_type=_f32) * dinv[:, None]


def _mid_body(p0_ref, p1_ref, hp_ref, dv_ref, b1_ref, w2_ref, o_ref):
    dinv = dv_ref[:, 0]
    y = (p0_ref[...] + p1_ref[...] + hp_ref[...]) * dinv[:, None] + b1_ref[...]
    y = jnp.maximum(y, 0.0)
    o_ref[...] = jnp.dot(y, w2_ref[...],
                         preferred_element_type=_f32) * dinv[:, None]


def _final_body(q0_ref, q1_ref, hp_ref, dv_ref, b2_ref, bt_ref,
                wfc_ref, bfc_ref, o_ref, psum, csum):
    i = pl.program_id(0)

    @pl.when(i == 0)
    def _():
        psum[...] = jnp.zeros_like(psum)
        csum[...] = jnp.zeros_like(csum)

    dinv = dv_ref[:, 0]
    h = (q0_ref[...] + q1_ref[...] + hp_ref[...]) * dinv[:, None] + b2_ref[...]
    rows = i * R + lax.broadcasted_iota(jnp.int32, (R, 1), 0)
    valid = rows < N
    h = jnp.where(valid, h, 0.0)
    bt = bt_ref[0, 0, :][:, None]
    gid = lax.broadcasted_iota(jnp.int32, (1, G), 1)
    onehot = (bt == gid).astype(_f32) * valid.astype(_f32)
    psum[...] += lax.dot_general(onehot, h, (((0,), (0,)), ((), ())),
                                 preferred_element_type=_f32)
    csum[...] += jnp.sum(onehot, axis=0)[:, None]
    pooled = psum[...] / jnp.maximum(csum[...], 1.0)
    o_ref[...] = jnp.dot(pooled, wfc_ref[...],
                         preferred_element_type=_f32) + bfc_ref[...]


def _row_spec(off):
    return pl.BlockSpec((R, D), lambda i, off=off: (i + off, 0))


def _deg_spec(off):
    return pl.BlockSpec((1, 1, R), lambda i, off=off: (i + off, 0, 0))


def _full_spec(shape):
    nd = len(shape)
    return pl.BlockSpec(shape, lambda i: (0,) * nd)


def _dinv_spec():
    return pl.BlockSpec((R, DEGOUT), lambda i: (i, 0))


def _prep_call(x, W1, deg2):
    return pl.pallas_call(
        _prep_body,
        grid=(NBLK,),
        in_specs=[_row_spec(0), _full_spec((D, D)), _deg_spec(0), _deg_spec(NBLK)],
        out_specs=(_row_spec(0), _dinv_spec()),
        out_shape=(jax.ShapeDtypeStruct((N, D), _f32),
                   jax.ShapeDtypeStruct((NP, DEGOUT), _f32)),
    )(x, W1, deg2, deg2)


def _mid_call(P, hp1, dinv, b1, W2):
    return pl.pallas_call(
        _mid_body,
        grid=(NBLK,),
        in_specs=[_row_spec(0), _row_spec(NBLK), _row_spec(0),
                  _dinv_spec(),
                  _full_spec((1, D)), _full_spec((D, D))],
        out_specs=_row_spec(0),
        out_shape=jax.ShapeDtypeStruct((N, D), _f32),
    )(P, P, hp1, dinv, b1, W2)


def _final_call(Q, hp2, dinv, b2, batch3, Wfc, bfc):
    return pl.pallas_call(
        _final_body,
        grid=(NBLK,),
        in_specs=[_row_spec(0), _row_spec(NBLK), _row_spec(0),
                  _dinv_spec(),
                  _full_spec((1, D)),
                  pl.BlockSpec((1, 1, R), lambda i: (i, 0, 0)),
                  _full_spec((D, D)), _full_spec((1, D))],
        out_specs=_full_spec((G, D)),
        out_shape=jax.ShapeDtypeStruct((G, D), _f32),
        scratch_shapes=[pltpu.VMEM((G, D), _f32), pltpu.VMEM((G, D), _f32)],
    )(Q, Q, hp2, dinv, b2, batch3, Wfc, bfc)


def kernel(x, edge_index, edge_attr, batch, W1, b1, W2, b2, Wfc, bfc):
    E = edge_index.shape[1]
    pad = EP - E
    ar = jnp.arange(pad, dtype=jnp.int32)
    src_pad = jnp.concatenate([edge_index[0], ar % 8192])
    dst_pad = jnp.concatenate([edge_index[1], N + (ar % CH)])
    batch3 = jnp.concatenate(
        [batch, jnp.full((NP - N,), G, jnp.int32)]).reshape(NBLK, 1, R)

    deg2 = _deg_call(dst_pad).reshape(NC * NBLK, 1, R)
    hp1, dinv = _prep_call(x, W1, deg2)
    P = _agg_call(src_pad, dst_pad, hp1)
    hp2 = _mid_call(P, hp1, dinv, b1.reshape(1, D), W2)
    Q = _agg_call(src_pad, dst_pad, hp2)
    return _final_call(Q, hp2, dinv, b2.reshape(1, D), batch3, Wfc,
                       bfc.reshape(1, D))


# speedup vs baseline: 1.4633x; 1.0230x over previous
import functools

import jax
import jax.numpy as jnp
from jax import lax
from jax.experimental import pallas as pl
from jax.experimental.pallas import tpu as pltpu
from jax.experimental.pallas import tpu_sc as plsc

N = 10000
D = 128
G = 64
NP = 10240
R = 640
NBLK = NP // R
NC, NS = 2, 16
RPT = NP // NS
CH = 128
NCHUNK = 80
PER_TILE = NCHUNK * CH
EP = NC * NS * PER_TILE
SEG = PER_TILE // 8
DEGOUT = 16

_f32 = jnp.float32


@functools.cache
def _mesh():
    return plsc.VectorSubcoreMesh(core_axis_name="c", subcore_axis_name="s",
                                  num_cores=NC, num_subcores=NS)


def _deg_call(edge_index):
    E_EDGES = edge_index.shape[1]
    assert E_EDGES % SEG == 0 and E_EDGES <= NC * NS * PER_TILE
    def body(dst_hbm, out_hbm, idx, hist, red, outv, staging):
        cid = lax.axis_index("c")
        sid = lax.axis_index("s")
        ones = jnp.ones((16,), _f32)
        zeros = jnp.zeros((16,), _f32)

        def zstep(t, c):
            hist[pl.ds(t * 16, 16)] = zeros
            return c

        lax.fori_loop(0, NP // 16, zstep, 0)
        tbase = (cid * NS + sid) * PER_TILE
        for s in range(PER_TILE // SEG):
            @pl.when(tbase + (s + 1) * SEG <= E_EDGES)
            def _(s=s):
                pltpu.sync_copy(
                    dst_hbm.at[1, pl.ds(tbase + s * SEG, SEG)],
                    idx.at[pl.ds(s * SEG, SEG)])

        nt = jnp.clip((E_EDGES - tbase) // CH, 0, NCHUNK)

        def hstep(t, c):
            iv = idx[pl.ds(t * 16, 16)]
            plsc.addupdate_scatter(hist, [iv], ones)
            return c

        lax.fori_loop(0, nt * (CH // 16), hstep, 0)
        pltpu.sync_copy(hist, staging.at[sid])
        plsc.subcore_barrier()
        pltpu.sync_copy(staging.at[:, pl.ds(sid * RPT, RPT)], red)
        for j in range(RPT // 16):
            s = red[0, j * 16:(j + 1) * 16]
            for r in range(1, NS):
                s = s + red[r, j * 16:(j + 1) * 16]
            outv[j * 16:(j + 1) * 16] = s
        pltpu.sync_copy(outv, out_hbm.at[pl.ds(cid * NP + sid * RPT, RPT)])

    return pl.kernel(
        body,
        out_type=jax.ShapeDtypeStruct((NC * NP,), _f32),
        mesh=_mesh(),
        compiler_params=pltpu.CompilerParams(needs_layout_passes=False),
        scratch_types=[
            pltpu.VMEM((PER_TILE,), jnp.int32),
            pltpu.VMEM((NP,), _f32),
            pltpu.VMEM((NS, RPT), _f32),
            pltpu.VMEM((RPT,), _f32),
            pltpu.VMEM_SHARED((NS, NP), _f32),
        ],
    )(edge_index)


def _agg_call(edge_index, hp):
    E_EDGES = edge_index.shape[1]
    assert E_EDGES % CH == 0 and E_EDGES <= NC * NS * PER_TILE

    def body(ei_hbm, hp_hbm, out_hbm,
             sb0, sb1, db0, db1, rb0, rb1, zb, acc, sg0, sg1, ss0, ss1):
        sb = (sb0, sb1)
        db = (db0, db1)
        rb = (rb0, rb1)
        sg = (sg0, sg1)
        ss = (ss0, ss1)
        cid = lax.axis_index("c")
        sid = lax.axis_index("s")
        for r in range(16):
            for c in range(8):
                zb[r, c * 16:(c + 1) * 16] = jnp.zeros((16,), _f32)
        rows0 = sid * RPT
        for j in range(RPT // 16):
            pltpu.sync_copy(zb, acc.at[pl.ds(rows0 + j * 16, 16)])
        plsc.subcore_barrier()
        tbase = (cid * NS + sid) * PER_TILE

        nt = jnp.clip((E_EDGES - tbase) // CH, 0, NCHUNK)

        def gstart(k, t):
            pltpu.sync_copy(ei_hbm.at[0, pl.ds(tbase + t * CH, CH)], sb[k])
            pltpu.async_copy(hp_hbm.at[sb[k]], rb[k], sg[k])

        def gwait(k):
            pltpu.make_async_copy(hp_hbm.at[sb[k]], rb[k], sg[k]).wait()

        def sstart(k, t):
            pltpu.sync_copy(ei_hbm.at[1, pl.ds(tbase + t * CH, CH)], db[k])
            pltpu.make_async_copy(rb[k], acc.at[db[k]], ss[k]).start(add=True)

        def swait(k):
            pltpu.make_async_copy(rb[k], acc.at[db[k]], ss[k]).wait()

        gstart(0, 0)
        gstart(1, 1)

        def pair(p, c):
            gwait(0)
            sstart(0, 2 * p)
            gwait(1)
            sstart(1, 2 * p + 1)

            @pl.when(p < nt // 2 - 1)
            def _():
                swait(0)
                gstart(0, 2 * p + 2)
                swait(1)
                gstart(1, 2 * p + 3)

            return c

        lax.fori_loop(0, nt // 2, pair, 0)
        swait(0)
        swait(1)
        plsc.subcore_barrier()
        pltpu.sync_copy(acc.at[pl.ds(rows0, RPT)],
                        out_hbm.at[pl.ds(cid * NP + rows0, RPT)])

    return pl.kernel(
        body,
        out_type=jax.ShapeDtypeStruct((NC * NP, D), _f32),
        mesh=_mesh(),
        scratch_types=(
            [pltpu.VMEM((CH,), jnp.int32)] * 4
            + [pltpu.VMEM((CH, D), _f32)] * 2
            + [pltpu.VMEM((16, D), _f32),
               pltpu.VMEM_SHARED((NP, D), _f32)]
            + [pltpu.SemaphoreType.DMA] * 4
        ),
    )(edge_index, hp)


def _prep_body(x_ref, w_ref, d0_ref, d1_ref, o_ref, dv_ref):
    dinv = lax.rsqrt(d0_ref[0, 0, :] + d1_ref[0, 0, :] + 1.0)
    dv_ref[...] = dinv[:, None] * jnp.ones((1, DEGOUT), _f32)
    o_ref[...] = jnp.dot(x_ref[...], w_ref[...],
                         preferred_element_type=_f32) * dinv[:, None]


def _mid_body(p0_ref, p1_ref, hp_ref, dv_ref, b1_ref, w2_ref, o_ref):
    dinv = dv_ref[:, 0]
    y = (p0_ref[...] + p1_ref[...] + hp_ref[...]) * dinv[:, None] + b1_ref[...]
    y = jnp.maximum(y, 0.0)
    o_ref[...] = jnp.dot(y, w2_ref[...],
                         preferred_element_type=_f32) * dinv[:, None]


def _final_body(q0_ref, q1_ref, hp_ref, dv_ref, b2_ref, bt_ref,
                wfc_ref, bfc_ref, o_ref, psum, csum):
    i = pl.program_id(0)

    @pl.when(i == 0)
    def _():
        psum[...] = jnp.zeros_like(psum)
        csum[...] = jnp.zeros_like(csum)

    dinv = dv_ref[:, 0]
    h = (q0_ref[...] + q1_ref[...] + hp_ref[...]) * dinv[:, None] + b2_ref[...]
    rows = i * R + lax.broadcasted_iota(jnp.int32, (R, 1), 0)
    valid = rows < N
    h = jnp.where(valid, h, 0.0)
    bt = bt_ref[0, 0, :][:, None]
    gid = lax.broadcasted_iota(jnp.int32, (1, G), 1)
    onehot = (bt == gid).astype(_f32) * valid.astype(_f32)
    psum[...] += lax.dot_general(onehot, h, (((0,), (0,)), ((), ())),
                                 preferred_element_type=_f32)
    csum[...] += jnp.sum(onehot, axis=0)[:, None]
    pooled = psum[...] / jnp.maximum(csum[...], 1.0)
    o_ref[...] = jnp.dot(pooled, wfc_ref[...],
                         preferred_element_type=_f32) + bfc_ref[...]


def _row_spec(off):
    return pl.BlockSpec((R, D), lambda i, off=off: (i + off, 0))


def _deg_spec(off):
    return pl.BlockSpec((1, 1, R), lambda i, off=off: (i + off, 0, 0))


def _full_spec(shape):
    nd = len(shape)
    return pl.BlockSpec(shape, lambda i: (0,) * nd)


def _dinv_spec():
    return pl.BlockSpec((R, DEGOUT), lambda i: (i, 0))


def _prep_call(x, W1, deg2):
    return pl.pallas_call(
        _prep_body,
        grid=(NBLK,),
        in_specs=[_row_spec(0), _full_spec((D, D)), _deg_spec(0), _deg_spec(NBLK)],
        out_specs=(_row_spec(0), _dinv_spec()),
        out_shape=(jax.ShapeDtypeStruct((N, D), _f32),
                   jax.ShapeDtypeStruct((NP, DEGOUT), _f32)),
    )(x, W1, deg2, deg2)


def _mid_call(P, hp1, dinv, b1, W2):
    return pl.pallas_call(
        _mid_body,
        grid=(NBLK,),
        in_specs=[_row_spec(0), _row_spec(NBLK), _row_spec(0),
                  _dinv_spec(),
                  _full_spec((1, D)), _full_spec((D, D))],
        out_specs=_row_spec(0),
        out_shape=jax.ShapeDtypeStruct((N, D), _f32),
    )(P, P, hp1, dinv, b1, W2)


def _final_call(Q, hp2, dinv, b2, batch3, Wfc, bfc):
    return pl.pallas_call(
        _final_body,
        grid=(NBLK,),
        in_specs=[_row_spec(0), _row_spec(NBLK), _row_spec(0),
                  _dinv_spec(),
                  _full_spec((1, D)),
                  pl.BlockSpec((1, 1, R), lambda i: (i, 0, 0)),
                  _full_spec((D, D)), _full_spec((1, D))],
        out_specs=_full_spec((G, D)),
        out_shape=jax.ShapeDtypeStruct((G, D), _f32),
        scratch_shapes=[pltpu.VMEM((G, D), _f32), pltpu.VMEM((G, D), _f32)],
    )(Q, Q, hp2, dinv, b2, batch3, Wfc, bfc)


def kernel(x, edge_index, edge_attr, batch, W1, b1, W2, b2, Wfc, bfc):
    batch3 = jnp.concatenate(
        [batch, jnp.full((NP - N,), G, jnp.int32)]).reshape(NBLK, 1, R)

    deg2 = _deg_call(edge_index).reshape(NC * NBLK, 1, R)
    hp1, dinv = _prep_call(x, W1, deg2)
    P = _agg_call(edge_index, hp1)
    hp2 = _mid_call(P, hp1, dinv, b1.reshape(1, D), W2)
    Q = _agg_call(edge_index, hp2)
    return _final_call(Q, hp2, dinv, b2.reshape(1, D), batch3, Wfc,
                       bfc.reshape(1, D))
